# baseline (device time: 31846 ns/iter reference)
import jax
import jax.numpy as jnp
from jax import lax
from jax.experimental import pallas as pl
from jax.experimental.pallas import tpu as pltpu

N_LAYERS = 3
CY = 2
CX = 2


def kernel(x, Win0, Wout0, Win1, Wout1, Win2, Wout2):
    m, d_loc = x.shape
    _, h_loc = Win0.shape
    hc = h_loc // CY
    dc = d_loc // CX
    f32 = jnp.float32

    def body(x_ref, win0, wout0, win1, wout1, win2, wout2, out_ref,
             h_send, h_recv, x_send, x_recv,
             ss_y, rs_y, ss_x, rs_x):
        my_x = lax.axis_index("x")
        my_y = lax.axis_index("y")
        y_peer = (my_x, 1 - my_y)
        x_peer = (1 - my_x, my_y)

        barrier_sem = pltpu.get_barrier_semaphore()
        for nbr in (y_peer, x_peer):
            pl.semaphore_signal(
                barrier_sem, inc=1,
                device_id=nbr, device_id_type=pl.DeviceIdType.MESH,
            )
        pl.semaphore_wait(barrier_sem, 2)

        wins = [win0, win1, win2]
        wouts = [wout0, wout1, wout2]

        def y_rdma(l, c):
            return pltpu.make_async_remote_copy(
                src_ref=h_send.at[l, c], dst_ref=h_recv.at[l, c],
                send_sem=ss_y.at[l, c], recv_sem=rs_y.at[l, c],
                device_id=y_peer, device_id_type=pl.DeviceIdType.MESH,
            )

        def x_rdma(l, d):
            return pltpu.make_async_remote_copy(
                src_ref=x_send.at[l, d], dst_ref=x_recv.at[l, d],
                send_sem=ss_x.at[l, d], recv_sem=rs_x.at[l, d],
                device_id=x_peer, device_id_type=pl.DeviceIdType.MESH,
            )

        x0 = x_ref[...]
        xp = None
        x_rdmas = []
        for l in range(N_LAYERS):
            win = wins[l]

            if l == 0:
                y_rdmas = []
                hp_chunks = []
                for c in range(CY):
                    hp_c = jnp.dot(
                        x0, win[:, c * hc:(c + 1) * hc],
                        preferred_element_type=f32,
                    )
                    h_send[l, c] = hp_c.astype(jnp.bfloat16)
                    r = y_rdma(l, c)
                    r.start()
                    y_rdmas.append(r)
                    hp_chunks.append(hp_c)
            else:
                hp = jnp.dot(xp, win[...], preferred_element_type=f32)
                for d in range(CX):
                    x_rdmas[d].wait()
                    hp = hp + jnp.dot(
                        x_recv[l - 1, d].astype(f32),
                        win[d * dc:(d + 1) * dc, :],
                        preferred_element_type=f32,
                    )
                y_rdmas = []
                hp_chunks = []
                for c in range(CY):
                    hp_c = hp[:, c * hc:(c + 1) * hc]
                    h_send[l, c] = hp_c.astype(jnp.bfloat16)
                    r = y_rdma(l, c)
                    r.start()
                    y_rdmas.append(r)
                    hp_chunks.append(hp_c)

            wout = wouts[l]
            xp = None
            for c in range(CY):
                y_rdmas[c].wait()
                h_c = jnp.maximum(
                    hp_chunks[c] + h_recv[l, c].astype(f32), 0.0
                )
                p = jnp.dot(
                    h_c, wout[c * hc:(c + 1) * hc, :],
                    preferred_element_type=f32,
                )
                xp = p if xp is None else xp + p

            x_rdmas = []
            for d in range(CX):
                x_send[l, d] = xp[:, d * dc:(d + 1) * dc].astype(jnp.bfloat16)
                r = x_rdma(l, d)
                r.start()
                x_rdmas.append(r)

        for d in range(CX):
            x_rdmas[d].wait()
            out_ref[:, d * dc:(d + 1) * dc] = (
                xp[:, d * dc:(d + 1) * dc]
                + x_recv[N_LAYERS - 1, d].astype(f32)
            )

    return pl.pallas_call(
        body,
        out_shape=jax.ShapeDtypeStruct((m, d_loc), f32),
        in_specs=[pl.BlockSpec(memory_space=pltpu.VMEM)] * 7,
        out_specs=pl.BlockSpec(memory_space=pltpu.VMEM),
        scratch_shapes=[
            pltpu.VMEM((N_LAYERS, CY, m, hc), jnp.bfloat16),
            pltpu.VMEM((N_LAYERS, CY, m, hc), jnp.bfloat16),
            pltpu.VMEM((N_LAYERS, CX, m, dc), jnp.bfloat16),
            pltpu.VMEM((N_LAYERS, CX, m, dc), jnp.bfloat16),
            pltpu.SemaphoreType.DMA((N_LAYERS, CY)),
            pltpu.SemaphoreType.DMA((N_LAYERS, CY)),
            pltpu.SemaphoreType.DMA((N_LAYERS, CX)),
            pltpu.SemaphoreType.DMA((N_LAYERS, CX)),
        ],
        compiler_params=pltpu.CompilerParams(collective_id=0),
    )(x, Win0, Wout0, Win1, Wout1, Win2, Wout2)


# device time: 30825 ns/iter; 1.0331x vs baseline; 1.0331x over previous
import jax
import jax.numpy as jnp
from jax import lax
from jax.experimental import pallas as pl
from jax.experimental.pallas import tpu as pltpu

N_LAYERS = 3
CY = 2
CX = 2


def kernel(x, Win0, Wout0, Win1, Wout1, Win2, Wout2):
    m, d_loc = x.shape
    _, h_loc = Win0.shape
    bf16 = jnp.bfloat16
    Win0, Wout0 = Win0.astype(bf16), Wout0.astype(bf16)
    Win1, Wout1 = Win1.astype(bf16), Wout1.astype(bf16)
    Win2, Wout2 = Win2.astype(bf16), Wout2.astype(bf16)
    hc = h_loc // CY
    dc = d_loc // CX
    f32 = jnp.float32

    def body(x_ref, win0, wout0, win1, wout1, win2, wout2, out_ref,
             h_send, h_recv, x_send, x_recv,
             ss_y, rs_y, ss_x, rs_x):
        my_x = lax.axis_index("x")
        my_y = lax.axis_index("y")
        y_peer = (my_x, 1 - my_y)
        x_peer = (1 - my_x, my_y)

        barrier_sem = pltpu.get_barrier_semaphore()
        for nbr in (y_peer, x_peer):
            pl.semaphore_signal(
                barrier_sem, inc=1,
                device_id=nbr, device_id_type=pl.DeviceIdType.MESH,
            )
        pl.semaphore_wait(barrier_sem, 2)

        wins = [win0, win1, win2]
        wouts = [wout0, wout1, wout2]

        def y_rdma(l, c):
            return pltpu.make_async_remote_copy(
                src_ref=h_send.at[l, c], dst_ref=h_recv.at[l, c],
                send_sem=ss_y.at[l, c], recv_sem=rs_y.at[l, c],
                device_id=y_peer, device_id_type=pl.DeviceIdType.MESH,
            )

        def x_rdma(l, d):
            return pltpu.make_async_remote_copy(
                src_ref=x_send.at[l, d], dst_ref=x_recv.at[l, d],
                send_sem=ss_x.at[l, d], recv_sem=rs_x.at[l, d],
                device_id=x_peer, device_id_type=pl.DeviceIdType.MESH,
            )

        x0 = x_ref[...].astype(jnp.bfloat16)
        xp = None
        xp_b = None
        x_rdmas = []
        for l in range(N_LAYERS):
            win = wins[l]

            if l == 0:
                y_rdmas = []
                hp_chunks = []
                for c in range(CY):
                    hp_c = jnp.dot(
                        x0, win[:, c * hc:(c + 1) * hc],
                        preferred_element_type=f32,
                    )
                    h_send[l, c] = hp_c.astype(jnp.bfloat16)
                    r = y_rdma(l, c)
                    r.start()
                    y_rdmas.append(r)
                    hp_chunks.append(hp_c)
            else:
                hp = jnp.dot(xp_b, win[...], preferred_element_type=f32)
                for d in range(CX):
                    x_rdmas[d].wait()
                    hp = hp + jnp.dot(
                        x_recv[l - 1, d],
                        win[d * dc:(d + 1) * dc, :],
                        preferred_element_type=f32,
                    )
                y_rdmas = []
                hp_chunks = []
                for c in range(CY):
                    hp_c = hp[:, c * hc:(c + 1) * hc]
                    h_send[l, c] = hp_c.astype(jnp.bfloat16)
                    r = y_rdma(l, c)
                    r.start()
                    y_rdmas.append(r)
                    hp_chunks.append(hp_c)

            wout = wouts[l]
            xp = None
            for c in range(CY):
                y_rdmas[c].wait()
                h_c = jnp.maximum(
                    hp_chunks[c] + h_recv[l, c].astype(f32), 0.0
                ).astype(jnp.bfloat16)
                p = jnp.dot(
                    h_c, wout[c * hc:(c + 1) * hc, :],
                    preferred_element_type=f32,
                )
                xp = p if xp is None else xp + p

            xp_b = xp.astype(jnp.bfloat16)
            x_rdmas = []
            for d in range(CX):
                x_send[l, d] = xp_b[:, d * dc:(d + 1) * dc]
                r = x_rdma(l, d)
                r.start()
                x_rdmas.append(r)

        for d in range(CX):
            x_rdmas[d].wait()
            out_ref[:, d * dc:(d + 1) * dc] = (
                xp[:, d * dc:(d + 1) * dc]
                + x_recv[N_LAYERS - 1, d].astype(f32)
            )

    return pl.pallas_call(
        body,
        out_shape=jax.ShapeDtypeStruct((m, d_loc), f32),
        in_specs=[pl.BlockSpec(memory_space=pltpu.VMEM)] * 7,
        out_specs=pl.BlockSpec(memory_space=pltpu.VMEM),
        scratch_shapes=[
            pltpu.VMEM((N_LAYERS, CY, m, hc), jnp.bfloat16),
            pltpu.VMEM((N_LAYERS, CY, m, hc), jnp.bfloat16),
            pltpu.VMEM((N_LAYERS, CX, m, dc), jnp.bfloat16),
            pltpu.VMEM((N_LAYERS, CX, m, dc), jnp.bfloat16),
            pltpu.SemaphoreType.DMA((N_LAYERS, CY)),
            pltpu.SemaphoreType.DMA((N_LAYERS, CY)),
            pltpu.SemaphoreType.DMA((N_LAYERS, CX)),
            pltpu.SemaphoreType.DMA((N_LAYERS, CX)),
        ],
        compiler_params=pltpu.CompilerParams(collective_id=0),
    )(x, Win0, Wout0, Win1, Wout1, Win2, Wout2)


# device time: 30794 ns/iter; 1.0342x vs baseline; 1.0010x over previous
import jax
import jax.numpy as jnp
from jax import lax
from jax.experimental import pallas as pl
from jax.experimental.pallas import tpu as pltpu

N_LAYERS = 3
CY = 2
CX = 2


def kernel(x, Win0, Wout0, Win1, Wout1, Win2, Wout2):
    m, d_loc = x.shape
    _, h_loc = Win0.shape
    hc = h_loc // CY
    dc = d_loc // CX
    f32 = jnp.float32
    bf16 = jnp.bfloat16
    Win0, Wout0 = Win0.astype(bf16), Wout0.astype(bf16)
    Win1, Wout1 = Win1.astype(bf16), Wout1.astype(bf16)
    Win2, Wout2 = Win2.astype(bf16), Wout2.astype(bf16)

    def body(x_ref, win0, wout0, win1, wout1, win2, wout2, out_ref,
             h_send, h_recv, x_send, x_recv,
             ss_y, rs_y, ss_x, rs_x):
        my_x = lax.axis_index("x")
        my_y = lax.axis_index("y")
        y_peer = (my_x, 1 - my_y)
        x_peer = (1 - my_x, my_y)

        barrier_sem = pltpu.get_barrier_semaphore()
        for nbr in (y_peer, x_peer):
            pl.semaphore_signal(
                barrier_sem, inc=1,
                device_id=nbr, device_id_type=pl.DeviceIdType.MESH,
            )
        pl.semaphore_wait(barrier_sem, 2)

        wins = [win0, win1, win2]
        wouts = [wout0, wout1, wout2]
        all_rdmas = []

        def y_rdma(l, c):
            r = pltpu.make_async_remote_copy(
                src_ref=h_send.at[l, c], dst_ref=h_recv.at[l, c],
                send_sem=ss_y.at[l, c], recv_sem=rs_y.at[l, c],
                device_id=y_peer, device_id_type=pl.DeviceIdType.MESH,
            )
            all_rdmas.append(r)
            return r

        def x_rdma(l, d):
            r = pltpu.make_async_remote_copy(
                src_ref=x_send.at[l, d], dst_ref=x_recv.at[l, d],
                send_sem=ss_x.at[l, d], recv_sem=rs_x.at[l, d],
                device_id=x_peer, device_id_type=pl.DeviceIdType.MESH,
            )
            all_rdmas.append(r)
            return r

        x0 = x_ref[...].astype(bf16)
        xp = None
        x_rdmas = []
        for l in range(N_LAYERS):
            win = wins[l]

            y_rdmas = []
            hp_chunks = []
            if l == 0:
                for c in range(CY):
                    hp_c = jnp.dot(
                        x0, win[:, c * hc:(c + 1) * hc],
                        preferred_element_type=f32,
                    )
                    h_send[l, c] = hp_c.astype(bf16)
                    r = y_rdma(l, c)
                    r.start()
                    y_rdmas.append(r)
                    hp_chunks.append(hp_c)
            else:
                xp_b = xp.astype(bf16)
                own = [
                    jnp.dot(
                        xp_b, win[:, c * hc:(c + 1) * hc],
                        preferred_element_type=f32,
                    )
                    for c in range(CY)
                ]
                recvs = []
                for d in range(CX):
                    x_rdmas[d].wait_recv()
                    recvs.append(x_recv[l - 1, d])
                for c in range(CY):
                    hp_c = own[c]
                    for d in range(CX):
                        hp_c = hp_c + jnp.dot(
                            recvs[d],
                            win[d * dc:(d + 1) * dc, c * hc:(c + 1) * hc],
                            preferred_element_type=f32,
                        )
                    h_send[l, c] = hp_c.astype(bf16)
                    r = y_rdma(l, c)
                    r.start()
                    y_rdmas.append(r)
                    hp_chunks.append(hp_c)

            wout = wouts[l]
            xp = None
            for c in range(CY):
                y_rdmas[c].wait_recv()
                h_c = jnp.maximum(
                    hp_chunks[c] + h_recv[l, c].astype(f32), 0.0
                ).astype(bf16)
                p = jnp.dot(
                    h_c, wout[c * hc:(c + 1) * hc, :],
                    preferred_element_type=f32,
                )
                xp = p if xp is None else xp + p

            xp_b16 = xp.astype(bf16)
            x_rdmas = []
            for d in range(CX):
                x_send[l, d] = xp_b16[:, d * dc:(d + 1) * dc]
                r = x_rdma(l, d)
                r.start()
                x_rdmas.append(r)

        for d in range(CX):
            x_rdmas[d].wait_recv()
            out_ref[:, d * dc:(d + 1) * dc] = (
                xp[:, d * dc:(d + 1) * dc]
                + x_recv[N_LAYERS - 1, d].astype(f32)
            )

        for r in all_rdmas:
            r.wait_send()

    return pl.pallas_call(
        body,
        out_shape=jax.ShapeDtypeStruct((m, d_loc), f32),
        in_specs=[pl.BlockSpec(memory_space=pltpu.VMEM)] * 7,
        out_specs=pl.BlockSpec(memory_space=pltpu.VMEM),
        scratch_shapes=[
            pltpu.VMEM((N_LAYERS, CY, m, hc), bf16),
            pltpu.VMEM((N_LAYERS, CY, m, hc), bf16),
            pltpu.VMEM((N_LAYERS, CX, m, dc), bf16),
            pltpu.VMEM((N_LAYERS, CX, m, dc), bf16),
            pltpu.SemaphoreType.DMA((N_LAYERS, CY)),
            pltpu.SemaphoreType.DMA((N_LAYERS, CY)),
            pltpu.SemaphoreType.DMA((N_LAYERS, CX)),
            pltpu.SemaphoreType.DMA((N_LAYERS, CX)),
        ],
        compiler_params=pltpu.CompilerParams(collective_id=0),
    )(x, Win0, Wout0, Win1, Wout1, Win2, Wout2)


# device time: 29040 ns/iter; 1.0966x vs baseline; 1.0604x over previous
import jax
import jax.numpy as jnp
from jax import lax
from jax.experimental import pallas as pl
from jax.experimental.pallas import tpu as pltpu

N_LAYERS = 3
CY = 2
CX = 2


def kernel(x, Win0, Wout0, Win1, Wout1, Win2, Wout2):
    m, d_loc = x.shape
    _, h_loc = Win0.shape
    hc = h_loc // CY
    dc = d_loc // CX
    f32 = jnp.float32
    bf16 = jnp.bfloat16
    wins_cat = jnp.concatenate(
        [Win0.astype(bf16), Win1.astype(bf16), Win2.astype(bf16)], axis=0
    )
    wouts_cat = jnp.concatenate(
        [Wout0.astype(bf16), Wout1.astype(bf16), Wout2.astype(bf16)], axis=0
    )

    def body(x_ref, wins, wouts, out_ref,
             h_send, h_recv, x_send, x_recv,
             ss_y, rs_y, ss_x, rs_x):
        my_x = lax.axis_index("x")
        my_y = lax.axis_index("y")
        y_peer = (my_x, 1 - my_y)
        x_peer = (1 - my_x, my_y)

        barrier_sem = pltpu.get_barrier_semaphore()
        for nbr in (y_peer, x_peer):
            pl.semaphore_signal(
                barrier_sem, inc=1,
                device_id=nbr, device_id_type=pl.DeviceIdType.MESH,
            )
        pl.semaphore_wait(barrier_sem, 2)

        all_rdmas = []

        def y_rdma(l, c):
            r = pltpu.make_async_remote_copy(
                src_ref=h_send.at[l, c], dst_ref=h_recv.at[l, c],
                send_sem=ss_y.at[l, c], recv_sem=rs_y.at[l, c],
                device_id=y_peer, device_id_type=pl.DeviceIdType.MESH,
            )
            all_rdmas.append(r)
            return r

        def x_rdma(l, d):
            r = pltpu.make_async_remote_copy(
                src_ref=x_send.at[l, d], dst_ref=x_recv.at[l, d],
                send_sem=ss_x.at[l, d], recv_sem=rs_x.at[l, d],
                device_id=x_peer, device_id_type=pl.DeviceIdType.MESH,
            )
            all_rdmas.append(r)
            return r

        x0 = x_ref[...].astype(bf16)
        xp = None
        x_rdmas = []
        for l in range(N_LAYERS):
            w0 = l * d_loc

            y_rdmas = []
            hp_chunks = []
            if l == 0:
                for c in range(CY):
                    hp_c = jnp.dot(
                        x0, wins[w0:w0 + d_loc, c * hc:(c + 1) * hc],
                        preferred_element_type=f32,
                    )
                    h_send[l, c] = hp_c.astype(bf16)
                    r = y_rdma(l, c)
                    r.start()
                    y_rdmas.append(r)
                    hp_chunks.append(hp_c)
            else:
                xp_b = xp.astype(bf16)
                own = [
                    jnp.dot(
                        xp_b, wins[w0:w0 + d_loc, c * hc:(c + 1) * hc],
                        preferred_element_type=f32,
                    )
                    for c in range(CY)
                ]
                recvs = []
                for d in range(CX):
                    x_rdmas[d].wait_recv()
                    recvs.append(x_recv[l - 1, d])
                for c in range(CY):
                    hp_c = own[c]
                    for d in range(CX):
                        hp_c = hp_c + jnp.dot(
                            recvs[d],
                            wins[w0 + d * dc:w0 + (d + 1) * dc,
                                 c * hc:(c + 1) * hc],
                            preferred_element_type=f32,
                        )
                    h_send[l, c] = hp_c.astype(bf16)
                    r = y_rdma(l, c)
                    r.start()
                    y_rdmas.append(r)
                    hp_chunks.append(hp_c)

            v0 = l * h_loc
            xp = None
            for c in range(CY):
                y_rdmas[c].wait_recv()
                h_c = jnp.maximum(
                    hp_chunks[c] + h_recv[l, c].astype(f32), 0.0
                ).astype(bf16)
                p = jnp.dot(
                    h_c, wouts[v0 + c * hc:v0 + (c + 1) * hc, :],
                    preferred_element_type=f32,
                )
                xp = p if xp is None else xp + p

            xp_b16 = xp.astype(bf16)
            x_rdmas = []
            for d in range(CX):
                x_send[l, d] = xp_b16[:, d * dc:(d + 1) * dc]
                r = x_rdma(l, d)
                r.start()
                x_rdmas.append(r)

        for d in range(CX):
            x_rdmas[d].wait_recv()
            out_ref[:, d * dc:(d + 1) * dc] = (
                xp[:, d * dc:(d + 1) * dc]
                + x_recv[N_LAYERS - 1, d].astype(f32)
            )

        for r in all_rdmas:
            r.wait_send()

    return pl.pallas_call(
        body,
        out_shape=jax.ShapeDtypeStruct((m, d_loc), f32),
        in_specs=[pl.BlockSpec(memory_space=pltpu.VMEM)] * 3,
        out_specs=pl.BlockSpec(memory_space=pltpu.VMEM),
        scratch_shapes=[
            pltpu.VMEM((N_LAYERS, CY, m, hc), bf16),
            pltpu.VMEM((N_LAYERS, CY, m, hc), bf16),
            pltpu.VMEM((N_LAYERS, CX, m, dc), bf16),
            pltpu.VMEM((N_LAYERS, CX, m, dc), bf16),
            pltpu.SemaphoreType.DMA((N_LAYERS, CY)),
            pltpu.SemaphoreType.DMA((N_LAYERS, CY)),
            pltpu.SemaphoreType.DMA((N_LAYERS, CX)),
            pltpu.SemaphoreType.DMA((N_LAYERS, CX)),
        ],
        compiler_params=pltpu.CompilerParams(collective_id=0),
    )(x, wins_cat, wouts_cat)


# device time: 27168 ns/iter; 1.1722x vs baseline; 1.0689x over previous
import jax
import jax.numpy as jnp
from jax import lax
from jax.experimental import pallas as pl
from jax.experimental.pallas import tpu as pltpu

N_LAYERS = 3
CY = 2
CX = 2


def kernel(x, Win0, Wout0, Win1, Wout1, Win2, Wout2):
    m, d_loc = x.shape
    _, h_loc = Win0.shape
    hc = h_loc // CY
    dc = d_loc // CX
    f32 = jnp.float32
    bf16 = jnp.bfloat16
    wins_cat = jnp.concatenate([Win0, Win1, Win2], axis=0)
    wouts_cat = jnp.concatenate([Wout0, Wout1, Wout2], axis=0)

    def body(x_ref, wins, wouts, out_ref,
             h_send, h_recv, x_send, x_recv,
             ss_y, rs_y, ss_x, rs_x):
        my_x = lax.axis_index("x")
        my_y = lax.axis_index("y")
        y_peer = (my_x, 1 - my_y)
        x_peer = (1 - my_x, my_y)

        barrier_sem = pltpu.get_barrier_semaphore()
        for nbr in (y_peer, x_peer):
            pl.semaphore_signal(
                barrier_sem, inc=1,
                device_id=nbr, device_id_type=pl.DeviceIdType.MESH,
            )
        pl.semaphore_wait(barrier_sem, 2)

        all_rdmas = []

        def y_rdma(l, c):
            r = pltpu.make_async_remote_copy(
                src_ref=h_send.at[l, c], dst_ref=h_recv.at[l, c],
                send_sem=ss_y.at[l, c], recv_sem=rs_y.at[l, c],
                device_id=y_peer, device_id_type=pl.DeviceIdType.MESH,
            )
            all_rdmas.append(r)
            return r

        def x_rdma(l, d):
            r = pltpu.make_async_remote_copy(
                src_ref=x_send.at[l, d], dst_ref=x_recv.at[l, d],
                send_sem=ss_x.at[l, d], recv_sem=rs_x.at[l, d],
                device_id=x_peer, device_id_type=pl.DeviceIdType.MESH,
            )
            all_rdmas.append(r)
            return r

        x0 = x_ref[...]
        xp = None
        x_rdmas = []
        for l in range(N_LAYERS):
            w0 = l * d_loc

            y_rdmas = []
            hp_chunks = []
            if l == 0:
                for c in range(CY):
                    hp_c = jnp.dot(
                        x0, wins[w0:w0 + d_loc, c * hc:(c + 1) * hc],
                        preferred_element_type=f32,
                    )
                    h_send[l, c] = hp_c.astype(bf16)
                    r = y_rdma(l, c)
                    r.start()
                    y_rdmas.append(r)
                    hp_chunks.append(hp_c)
            else:
                own = [
                    jnp.dot(
                        xp, wins[w0:w0 + d_loc, c * hc:(c + 1) * hc],
                        preferred_element_type=f32,
                    )
                    for c in range(CY)
                ]
                recvs = []
                for d in range(CX):
                    x_rdmas[d].wait_recv()
                    recvs.append(x_recv[l - 1, d].astype(f32))
                for c in range(CY):
                    hp_c = own[c]
                    for d in range(CX):
                        hp_c = hp_c + jnp.dot(
                            recvs[d],
                            wins[w0 + d * dc:w0 + (d + 1) * dc,
                                 c * hc:(c + 1) * hc],
                            preferred_element_type=f32,
                        )
                    h_send[l, c] = hp_c.astype(bf16)
                    r = y_rdma(l, c)
                    r.start()
                    y_rdmas.append(r)
                    hp_chunks.append(hp_c)

            v0 = l * h_loc
            xp = None
            for c in range(CY):
                y_rdmas[c].wait_recv()
                h_c = jnp.maximum(
                    hp_chunks[c] + h_recv[l, c].astype(f32), 0.0
                )
                p = jnp.dot(
                    h_c, wouts[v0 + c * hc:v0 + (c + 1) * hc, :],
                    preferred_element_type=f32,
                )
                xp = p if xp is None else xp + p

            xp_b16 = xp.astype(bf16)
            x_rdmas = []
            for d in range(CX):
                x_send[l, d] = xp_b16[:, d * dc:(d + 1) * dc]
                r = x_rdma(l, d)
                r.start()
                x_rdmas.append(r)

        for d in range(CX):
            x_rdmas[d].wait_recv()
            out_ref[:, d * dc:(d + 1) * dc] = (
                xp[:, d * dc:(d + 1) * dc]
                + x_recv[N_LAYERS - 1, d].astype(f32)
            )

        for r in all_rdmas:
            r.wait_send()

    return pl.pallas_call(
        body,
        out_shape=jax.ShapeDtypeStruct((m, d_loc), f32),
        in_specs=[pl.BlockSpec(memory_space=pltpu.VMEM)] * 3,
        out_specs=pl.BlockSpec(memory_space=pltpu.VMEM),
        scratch_shapes=[
            pltpu.VMEM((N_LAYERS, CY, m, hc), bf16),
            pltpu.VMEM((N_LAYERS, CY, m, hc), bf16),
            pltpu.VMEM((N_LAYERS, CX, m, dc), bf16),
            pltpu.VMEM((N_LAYERS, CX, m, dc), bf16),
            pltpu.SemaphoreType.DMA((N_LAYERS, CY)),
            pltpu.SemaphoreType.DMA((N_LAYERS, CY)),
            pltpu.SemaphoreType.DMA((N_LAYERS, CX)),
            pltpu.SemaphoreType.DMA((N_LAYERS, CX)),
        ],
        compiler_params=pltpu.CompilerParams(collective_id=0),
    )(x, wins_cat, wouts_cat)
